# baseline (speedup 1.0000x reference)
import functools

import jax
import jax.numpy as jnp
from jax import lax
from jax.experimental import pallas as pl
from jax.experimental.pallas import tpu as pltpu
from jax.experimental.pallas import tpu_sc as plsc

_INFO = plsc.get_sparse_core_info()
_NC = _INFO.num_cores
_NS = _INFO.num_subcores
_NW = _NC * _NS
_L = _INFO.num_lanes


def _make_gather(V, D, B):
    rps = 128 // D
    assert B % _NW == 0 and 128 % D == 0 and V % rps == 0
    b_per_w = B // _NW
    out_rows_w = b_per_w * D // 128
    n_chunks = b_per_w // _L
    mesh = plsc.VectorSubcoreMesh(core_axis_name="c", subcore_axis_name="s")

    @functools.partial(
        pl.kernel,
        mesh=mesh,
        out_type=jax.ShapeDtypeStruct((B * D // 128, 128), jnp.float32),
        scratch_types=[
            pltpu.VMEM((b_per_w,), jnp.int32),
            pltpu.VMEM((b_per_w,), jnp.int32),
            pltpu.VMEM((b_per_w, 128), jnp.float32),
            pltpu.VMEM((out_rows_w, 128), jnp.float32),
            pltpu.SemaphoreType.DMA,
        ],
        compiler_params=pltpu.CompilerParams(needs_layout_passes=False),
    )
    def k(table_hbm, idx_hbm, out_hbm, idx_v, sup_v, rows_v, out_v, sem):
        wid = lax.axis_index("s") * _NC + lax.axis_index("c")
        base = wid * b_per_w
        pltpu.sync_copy(idx_hbm.at[pl.ds(base, b_per_w)], idx_v)

        iota = lax.iota(jnp.int32, _L)

        def prep(c, _):
            iv = idx_v[pl.ds(c * _L, _L)]
            sup_v[pl.ds(c * _L, _L)] = lax.shift_right_logical(iv, 2)
            return _

        lax.fori_loop(0, n_chunks, prep, None)

        pltpu.async_copy(table_hbm.at[sup_v], rows_v, sem).wait()

        def extract(c, _):
            iv = idx_v[pl.ds(c * _L, _L)]
            colbase = (iv & 3) * D
            rowv = iota + c * _L
            flatbase = rowv * D
            for j in range(D):
                val = plsc.load_gather(rows_v, [rowv, colbase + j])
                flat = flatbase + j
                plsc.store_scatter(
                    out_v, [lax.shift_right_logical(flat, 7), flat & 127], val
                )
            return _

        lax.fori_loop(0, n_chunks, extract, None)

        pltpu.sync_copy(out_v, out_hbm.at[pl.ds(wid * out_rows_w, out_rows_w)])

    return k


@jax.jit
def kernel(idx_list, embs):
    B = idx_list.shape[0]
    V, D = embs.shape
    table = embs.reshape(V * D // 128, 128)
    out = _make_gather(V, D, B)(table, idx_list)
    return out.reshape(B, D)


# speedup vs baseline: 3.9654x; 3.9654x over previous
import functools

import jax
import jax.numpy as jnp
from jax import lax
from jax.experimental import pallas as pl
from jax.experimental.pallas import tpu as pltpu
from jax.experimental.pallas import tpu_sc as plsc

_INFO = plsc.get_sparse_core_info()
_NC = _INFO.num_cores
_NS = _INFO.num_subcores
_NW = _NC * _NS
_L = _INFO.num_lanes
_CH = 8


def _make_gather(V, D, B):
    assert B % (_NW * _L) == 0 and D % _L == 0
    b_per_w = B // _NW
    n_pairs = b_per_w // (2 * _CH)
    mesh = plsc.VectorSubcoreMesh(core_axis_name="c", subcore_axis_name="s")

    @functools.partial(
        pl.kernel,
        mesh=mesh,
        out_type=jax.ShapeDtypeStruct((D, B), jnp.float32),
        scratch_types=[
            pltpu.VMEM((b_per_w,), jnp.int32),
            pltpu.VMEM((_CH, D, 128), jnp.float32),
            pltpu.VMEM((_CH, D, 128), jnp.float32),
            pltpu.VMEM((D, b_per_w), jnp.float32),
            pltpu.SemaphoreType.DMA,
            pltpu.SemaphoreType.DMA,
        ],
        compiler_params=pltpu.CompilerParams(needs_layout_passes=False),
    )
    def k(table_hbm, idx_hbm, out_hbm, idx_v, bank0, bank1, gath_v, s0, s1):
        wid = lax.axis_index("s") * _NC + lax.axis_index("c")
        base = wid * b_per_w
        pltpu.sync_copy(idx_hbm.at[pl.ds(base, b_per_w)], idx_v)

        iota = lax.iota(jnp.int32, _L)

        def load_iv(p):
            return idx_v[pl.ds(p * 2 * _CH, _L)]

        def fire(iv, lane_base, bank, sem):
            for l in range(_CH):
                col_off = pl.multiple_of(iv[lane_base + l] & -128, 128)
                pltpu.async_copy(
                    table_hbm.at[:, pl.ds(col_off, 128)], bank.at[l], sem
                )

        def drain_extract(c, iv, lane_base, bank, sem):
            for l in range(_CH):
                pltpu.make_async_copy(
                    table_hbm.at[:, pl.ds(0, 128)], bank.at[l], sem
                ).wait()
            lane = iv & 127
            for l in range(_CH):
                lc = jnp.full((_L,), lane[lane_base + l], jnp.int32)
                lv = jnp.full((_L,), l, jnp.int32)
                jv = jnp.full((_L,), c * _CH + l, jnp.int32)
                for h in range(D // _L):
                    dvec = iota + h * _L
                    val = plsc.load_gather(bank, [lv, dvec, lc])
                    plsc.store_scatter(gath_v, [dvec, jv], val)

        fire(load_iv(0), 0, bank0, s0)

        def pair(p, _):
            ivp = load_iv(p)
            fire(ivp, _CH, bank1, s1)
            drain_extract(2 * p, ivp, 0, bank0, s0)
            ivn = load_iv(jnp.minimum(p + 1, n_pairs - 1))

            @pl.when(p + 1 < n_pairs)
            def _fire_next():
                fire(ivn, 0, bank0, s0)

            drain_extract(2 * p + 1, ivp, _CH, bank1, s1)
            return _

        lax.fori_loop(0, n_pairs, pair, None)

        pltpu.sync_copy(gath_v, out_hbm.at[:, pl.ds(base, b_per_w)])

    return k


@jax.jit
def kernel(idx_list, embs):
    B = idx_list.shape[0]
    V, D = embs.shape
    out_t = _make_gather(V, D, B)(embs.T, idx_list)
    return out_t.T
